# Initial kernel scaffold; baseline (speedup 1.0000x reference)
import jax
import jax.numpy as jnp
from jax.experimental import pallas as pl


def kernel(hidden_states, Wr, W1, W2):
    raise NotImplementedError("write your pallas kernel here")


# speedup vs baseline: 2.2611x; 2.2611x over previous
import functools
import jax
import jax.numpy as jnp
from jax.experimental import pallas as pl
from jax.experimental.pallas import tpu as pltpu

_BM = 128
_BI = 1024


def _router_body(x_ref, wr_ref, eid_ref, prob_ref, *, n_exp):
    logits = jnp.dot(x_ref[...], wr_ref[...], preferred_element_type=jnp.float32)
    col = jax.lax.broadcasted_iota(jnp.int32, logits.shape, 1)
    valid = col < n_exp
    neg = jnp.where(valid, logits, -1e30)
    m = jnp.max(neg, axis=1, keepdims=True)
    p = jnp.exp(neg - m)
    s = jnp.sum(p, axis=1, keepdims=True)
    probs = p / s
    pmax = jnp.max(probs, axis=1, keepdims=True)
    is_max = (probs == pmax) & valid
    eid_ref[...] = jnp.min(jnp.where(is_max, col, n_exp), axis=1, keepdims=True)
    prob_ref[...] = pmax


def _mlp_body(te_ref, tv_ref, x_ref, w1_ref, w2_ref, pr_ref, out_ref):
    i = pl.program_id(1)

    @pl.when(i == 0)
    def _init():
        out_ref[...] = jnp.zeros_like(out_ref)

    t = pl.program_id(0)

    @pl.when(tv_ref[t] == 1)
    def _compute():
        h1 = jnp.dot(x_ref[...], w1_ref[0], preferred_element_type=jnp.float32)
        act = 0.5 * h1 * (1.0 + jax.lax.erf(h1 * 0.7071067811865476))
        act = act * pr_ref[:, :1]
        out_ref[...] += jnp.dot(act, w2_ref[0], preferred_element_type=jnp.float32)


def kernel(hidden_states, Wr, W1, W2):
    B, S, H = hidden_states.shape
    E, _, I = W1.shape
    T = B * S
    hs = hidden_states.reshape(T, H)

    wr_pad = jnp.zeros((H, 128), jnp.float32).at[:, :E].set(Wr)
    eid2, prob2 = pl.pallas_call(
        functools.partial(_router_body, n_exp=E),
        out_shape=[
            jax.ShapeDtypeStruct((T, 1), jnp.int32),
            jax.ShapeDtypeStruct((T, 1), jnp.float32),
        ],
    )(hs, wr_pad)
    eid = eid2[:, 0]
    prob = prob2[:, 0]

    NT = T // _BM + E
    P = NT * _BM
    onehot = (eid[:, None] == jnp.arange(E)[None, :]).astype(jnp.int32)
    counts = onehot.sum(axis=0)
    rank = (jnp.cumsum(onehot, axis=0) - onehot)
    rank = (rank * onehot).sum(axis=1)
    padded = ((counts + _BM - 1) // _BM) * _BM
    ends = jnp.cumsum(padded)
    offsets = ends - padded
    pos = offsets[eid] + rank
    perm = jnp.zeros((P,), jnp.int32).at[pos].set(jnp.arange(T, dtype=jnp.int32))
    row0 = jnp.minimum(jnp.arange(NT, dtype=jnp.int32) * _BM, ends[-1] - 1)
    tile_expert = jnp.minimum(
        jnp.searchsorted(ends, row0, side="right").astype(jnp.int32), E - 1)
    tile_valid = (jnp.arange(NT, dtype=jnp.int32) * _BM < ends[-1]).astype(jnp.int32)

    x_sorted = jnp.take(hs, perm, axis=0)
    prob_sorted = jnp.take(prob, perm, axis=0)
    prob_b = jnp.broadcast_to(prob_sorted[:, None], (P, 128))

    NI = I // _BI
    grid_spec = pltpu.PrefetchScalarGridSpec(
        num_scalar_prefetch=2,
        grid=(NT, NI),
        in_specs=[
            pl.BlockSpec((_BM, H), lambda t, i, te, tv: (t, 0)),
            pl.BlockSpec((1, H, _BI), lambda t, i, te, tv: (te[t], 0, i)),
            pl.BlockSpec((1, _BI, H), lambda t, i, te, tv: (te[t], i, 0)),
            pl.BlockSpec((_BM, 128), lambda t, i, te, tv: (t, 0)),
        ],
        out_specs=pl.BlockSpec((_BM, H), lambda t, i, te, tv: (t, 0)),
    )
    out_sorted = pl.pallas_call(
        _mlp_body,
        grid_spec=grid_spec,
        out_shape=jax.ShapeDtypeStruct((P, H), jnp.float32),
    )(tile_expert, tile_valid, x_sorted, W1, W2, prob_b)

    out = jnp.take(out_sorted, pos, axis=0)
    return out.reshape(B, S, H)


# speedup vs baseline: 2.8519x; 1.2613x over previous
import functools
import jax
import jax.numpy as jnp
from jax.experimental import pallas as pl
from jax.experimental.pallas import tpu as pltpu

_BM = 128
_BI = 1024


def _router_body(x_ref, wr_ref, eid_ref, prob_ref, *, n_exp):
    logits = jnp.dot(x_ref[...], wr_ref[...], preferred_element_type=jnp.float32)
    col = jax.lax.broadcasted_iota(jnp.int32, logits.shape, 1)
    valid = col < n_exp
    neg = jnp.where(valid, logits, -1e30)
    m = jnp.max(neg, axis=1, keepdims=True)
    p = jnp.exp(neg - m)
    s = jnp.sum(p, axis=1, keepdims=True)
    probs = p / s
    pmax = jnp.max(probs, axis=1, keepdims=True)
    is_max = (probs == pmax) & valid
    eid_ref[...] = jnp.min(jnp.where(is_max, col, n_exp), axis=1, keepdims=True)
    prob_ref[...] = pmax


def _mlp_body(te_ref, tv_ref, x_ref, w1_ref, w2_ref, pr_ref, out_ref):
    t = pl.program_id(0)

    @pl.when(tv_ref[t] == 1)
    def _compute():
        h1 = jnp.dot(x_ref[...], w1_ref[0], preferred_element_type=jnp.float32)
        act = 0.5 * h1 * (1.0 + jax.lax.erf(h1 * 0.7071067811865476))
        act = act * pr_ref[:, :1]
        out_ref[...] = jnp.dot(act.astype(jnp.bfloat16), w2_ref[0],
                               preferred_element_type=jnp.float32)

    @pl.when(tv_ref[t] == 0)
    def _zero():
        out_ref[...] = jnp.zeros_like(out_ref)


def kernel(hidden_states, Wr, W1, W2):
    B, S, H = hidden_states.shape
    E, _, I = W1.shape
    T = B * S
    hs = hidden_states.reshape(T, H)

    wr_pad = jnp.zeros((H, 128), jnp.float32).at[:, :E].set(Wr)
    eid2, prob2 = pl.pallas_call(
        functools.partial(_router_body, n_exp=E),
        out_shape=[
            jax.ShapeDtypeStruct((T, 1), jnp.int32),
            jax.ShapeDtypeStruct((T, 1), jnp.float32),
        ],
    )(hs, wr_pad)
    eid = eid2[:, 0]
    prob = prob2[:, 0]

    NT = T // _BM + E
    P = NT * _BM
    onehot = (eid[:, None] == jnp.arange(E)[None, :]).astype(jnp.int32)
    counts = onehot.sum(axis=0)
    rank = (jnp.cumsum(onehot, axis=0) - onehot)
    rank = (rank * onehot).sum(axis=1)
    padded = ((counts + _BM - 1) // _BM) * _BM
    ends = jnp.cumsum(padded)
    offsets = ends - padded
    pos = offsets[eid] + rank
    perm = jnp.zeros((P,), jnp.int32).at[pos].set(jnp.arange(T, dtype=jnp.int32))
    row0 = jnp.minimum(jnp.arange(NT, dtype=jnp.int32) * _BM, ends[-1] - 1)
    tile_expert = jnp.minimum(
        jnp.searchsorted(ends, row0, side="right").astype(jnp.int32), E - 1)
    tile_valid = (jnp.arange(NT, dtype=jnp.int32) * _BM < ends[-1]).astype(jnp.int32)

    x_sorted = jnp.take(hs, perm, axis=0).astype(jnp.bfloat16)
    prob_sorted = jnp.take(prob, perm, axis=0)
    prob_b = jnp.broadcast_to(prob_sorted[:, None], (P, 128))

    grid_spec = pltpu.PrefetchScalarGridSpec(
        num_scalar_prefetch=2,
        grid=(NT,),
        in_specs=[
            pl.BlockSpec((_BM, H), lambda t, te, tv: (t, 0)),
            pl.BlockSpec((1, H, I), lambda t, te, tv: (te[t], 0, 0)),
            pl.BlockSpec((1, I, H), lambda t, te, tv: (te[t], 0, 0)),
            pl.BlockSpec((_BM, 128), lambda t, te, tv: (t, 0)),
        ],
        out_specs=pl.BlockSpec((_BM, H), lambda t, te, tv: (t, 0)),
    )
    out_sorted = pl.pallas_call(
        _mlp_body,
        grid_spec=grid_spec,
        out_shape=jax.ShapeDtypeStruct((P, H), jnp.float32),
    )(tile_expert, tile_valid, x_sorted,
      W1.astype(jnp.bfloat16), W2.astype(jnp.bfloat16), prob_b)

    out = jnp.take(out_sorted, pos, axis=0)
    return out.reshape(B, S, H)
